# baseline (speedup 1.0000x reference)
import jax
import jax.numpy as jnp
from jax import lax
from jax.experimental import pallas as pl
from jax.experimental.pallas import tpu as pltpu
from jax.experimental.pallas import tpu_sc as plsc

V = 50265
H = 768
B, P, T = 16, 8, 129
NSEQ = B + B * P
TOK = T - 1
NQ = 4
QT = TOK // NQ
SEQ_PER_TILE = NSEQ // 8
FV = H // 16


def _rsqrt_newton(x):
    i = lax.bitcast_convert_type(x, jnp.int32)
    i = jnp.int32(0x5F3759DF) - lax.shift_right_logical(i, 1)
    y = lax.bitcast_convert_type(i, jnp.float32)
    for _ in range(3):
        y = y * (jnp.float32(1.5) - jnp.float32(0.5) * x * y * y)
    return y


def _allreduce_sum(v, tmp_ref):
    lanes = lax.iota(jnp.int32, 16)
    for k in (8, 4, 2, 1):
        tmp_ref[...] = v
        v = v + plsc.load_gather(tmp_ref, [(lanes + k) & 15])
    return v


def _sc_body(ids_hbm, c_hbm, tab_hbm, out_hbm, ids_v, c_v, rows_v, acc_v, a_v,
             tmp_v, sem):
    wid = lax.axis_index("c") * 16 + lax.axis_index("s")
    q = wid // 8
    zero16 = jnp.zeros((16,), jnp.float32)

    pltpu.sync_copy(c_hbm.at[q], c_v)
    pltpu.sync_copy(ids_hbm.at[wid], ids_v)

    def chunk(j, carry):
        pltpu.async_copy(tab_hbm.at[ids_v.at[j]], rows_v, sem).wait()

        def tok_a(t, s_carry):
            def feat_a(i, c2):
                vs, vq = c2
                sl = pl.ds(i * 16, 16)
                e = rows_v[t, sl] + c_v[t, sl]
                rows_v[t, sl] = e
                return (vs + e, vq + e * e)

            vs, vq = lax.fori_loop(0, FV, feat_a, (zero16, zero16), unroll=8)
            mu = _allreduce_sum(vs, tmp_v) * jnp.float32(1.0 / H)
            var = (_allreduce_sum(vq, tmp_v) * jnp.float32(1.0 / H)
                   - mu * mu + jnp.float32(1e-5))
            w = _rsqrt_newton(var)
            a_v[t] = w
            return s_carry + mu * w

        s_mu_w = lax.fori_loop(0, QT, tok_a, zero16)

        def init_b(i, _):
            acc_v[j, pl.ds(i * 16, 16)] = -s_mu_w
            return 0

        lax.fori_loop(0, FV, init_b, 0, unroll=8)

        def tok_b(t, _):
            w = a_v[t]

            def feat_b(i, _):
                sl = pl.ds(i * 16, 16)
                plsc.addupdate(acc_v.at[j, sl], rows_v[t, sl] * w)
                return 0

            lax.fori_loop(0, FV, feat_b, 0, unroll=8)
            return 0

        lax.fori_loop(0, QT, tok_b, 0)
        return carry

    lax.fori_loop(0, SEQ_PER_TILE, chunk, 0)
    pltpu.sync_copy(acc_v, out_hbm.at[wid])


def _make_sc_pool():
    mesh = plsc.VectorSubcoreMesh(core_axis_name="c", subcore_axis_name="s")
    return pl.kernel(
        _sc_body,
        out_type=jax.ShapeDtypeStruct((32, SEQ_PER_TILE, H), jnp.float32),
        mesh=mesh,
        compiler_params=pltpu.CompilerParams(needs_layout_passes=False),
        scratch_types=[
            pltpu.VMEM((SEQ_PER_TILE, QT), jnp.int32),
            pltpu.VMEM((QT, H), jnp.float32),
            pltpu.VMEM((QT, H), jnp.float32),
            pltpu.VMEM((SEQ_PER_TILE, H), jnp.float32),
            pltpu.VMEM((QT, 16), jnp.float32),
            pltpu.VMEM((16,), jnp.float32),
            pltpu.SemaphoreType.DMA,
        ],
    )


def _tc_body(hp_ref, pp_ref, g_ref, bb_ref, w_ref, wb_ref, out_ref):
    inv = jnp.float32(1.0 / TOK)
    g = g_ref[...]
    bb = bb_ref[...]
    hp = hp_ref[...]
    pp = pp_ref[...]
    pooled_h = (hp[0] + hp[1] + hp[2] + hp[3]) * inv * g + bb
    pooled_p = (pp[0] + pp[1] + pp[2] + pp[3]) * inv * g + bb
    hist = lax.dot_general(pooled_h, w_ref[...], (((1,), (1,)), ((), ())),
                           precision=lax.Precision.HIGHEST,
                           preferred_element_type=jnp.float32)
    hist = hist + wb_ref[...]
    diff = pooled_p - hist[:, None, :]
    d2 = jnp.sum(diff * diff, axis=-1)
    feats = -jnp.sqrt(d2)
    m = jnp.max(feats, axis=-1, keepdims=True)
    ex = jnp.exp(feats - m)
    out_ref[...] = ex / jnp.sum(ex, axis=-1, keepdims=True)


def kernel(persona, history, word_emb, pos_emb, tok_type_emb, ln_g, ln_b, W, b):
    ids = jnp.concatenate(
        [history[:, 1:].reshape(B, TOK),
         persona[:, :, 1:].reshape(B * P, TOK)], axis=0).astype(jnp.int32)
    ids = ids.reshape(NSEQ, NQ, QT).transpose(1, 0, 2).reshape(32, SEQ_PER_TILE, QT)
    c = (pos_emb[2:2 + TOK] + tok_type_emb[0]).reshape(NQ, QT, H)

    partial = _make_sc_pool()(ids, c, word_emb)
    partial = partial.reshape(NQ, NSEQ, H)
    hp = partial[:, :B]
    pp = partial[:, B:].reshape(NQ, B, P, H)

    return pl.pallas_call(
        _tc_body,
        out_shape=jax.ShapeDtypeStruct((B, P), jnp.float32),
    )(hp, pp, ln_g, ln_b, W, b)


# speedup vs baseline: 1.6449x; 1.6449x over previous
import jax
import jax.numpy as jnp
from jax import lax
from jax.experimental import pallas as pl
from jax.experimental.pallas import tpu as pltpu
from jax.experimental.pallas import tpu_sc as plsc

V = 50265
H = 768
B, P, T = 16, 8, 129
NSEQ = B + B * P
TOK = T - 1
NQ = 4
QT = TOK // NQ
SEQ_PER_TILE = NSEQ // 8
FV = H // 16


def _rsqrt_newton(x):
    i = lax.bitcast_convert_type(x, jnp.int32)
    i = jnp.int32(0x5F3759DF) - lax.shift_right_logical(i, 1)
    y = lax.bitcast_convert_type(i, jnp.float32)
    for _ in range(3):
        y = y * (jnp.float32(1.5) - jnp.float32(0.5) * x * y * y)
    return y


def _sc_body(ids_hbm, c_hbm, tab_hbm, out_hbm, corr_hbm,
             ids_v, c_v, rows_v, acc_v, corr_v, stats_v, a_v, a_s, sem0, sem1):
    wid = lax.axis_index("c") * 16 + lax.axis_index("s")
    q = wid // 8
    zero16 = jnp.zeros((16,), jnp.float32)
    lanes = lax.iota(jnp.int32, 16)

    pltpu.sync_copy(ids_hbm.at[wid], ids_v)
    pltpu.sync_copy(c_hbm.at[q], c_v)
    pltpu.async_copy(tab_hbm.at[ids_v.at[0]], rows_v.at[0], sem0)

    def _compute(j, buf):
        def tok_a(t, _):
            vs0 = vs1 = vq0 = vq1 = zero16
            for i in range(FV):
                sl = pl.ds(i * 16, 16)
                e = rows_v[buf, t, sl] + c_v[t, sl]
                rows_v[buf, t, sl] = e
                if i % 2 == 0:
                    vs0 = vs0 + e
                    vq0 = vq0 + e * e
                else:
                    vs1 = vs1 + e
                    vq1 = vq1 + e * e
            stats_v[0, t] = vs0 + vs1
            stats_v[1, t] = vq0 + vq1
            return 0

        lax.fori_loop(0, QT, tok_a, 0)

        mw_total = zero16
        for h in range(2):
            tix = lanes + (h * 16)
            ss0 = ss1 = qq0 = qq1 = zero16
            for l in range(0, 16, 2):
                ss0 = ss0 + plsc.load_gather(
                    stats_v, [jnp.full((16,), 0, jnp.int32), tix,
                              jnp.full((16,), l, jnp.int32)])
                ss1 = ss1 + plsc.load_gather(
                    stats_v, [jnp.full((16,), 0, jnp.int32), tix,
                              jnp.full((16,), l + 1, jnp.int32)])
                qq0 = qq0 + plsc.load_gather(
                    stats_v, [jnp.full((16,), 1, jnp.int32), tix,
                              jnp.full((16,), l, jnp.int32)])
                qq1 = qq1 + plsc.load_gather(
                    stats_v, [jnp.full((16,), 1, jnp.int32), tix,
                              jnp.full((16,), l + 1, jnp.int32)])
            ss = ss0 + ss1
            qq = qq0 + qq1
            mu = ss * jnp.float32(1.0 / H)
            var = qq * jnp.float32(1.0 / H) - mu * mu + jnp.float32(1e-5)
            w = _rsqrt_newton(var)
            for l in range(16):
                a_s[h * 16 + l] = w[l]
            mw_total = mw_total + mu * w
        corr_v[j] = mw_total

        w0 = jnp.full((16,), a_s[0], jnp.float32)
        for i in range(FV):
            sl = pl.ds(i * 16, 16)
            acc_v[j, sl] = rows_v[buf, 0, sl] * w0

        def tok_b(t, _):
            w = jnp.full((16,), a_s[t], jnp.float32)
            for i in range(FV):
                sl = pl.ds(i * 16, 16)
                plsc.addupdate(acc_v.at[j, sl], rows_v[buf, t, sl] * w)
            return 0

        lax.fori_loop(1, QT, tok_b, 0)

    def pair(k, carry):
        j0 = 2 * k
        j1 = j0 + 1
        pltpu.async_copy(tab_hbm.at[ids_v.at[j1]], rows_v.at[1], sem1)
        pltpu.make_async_copy(
            tab_hbm.at[ids_v.at[j0]], rows_v.at[0], sem0).wait()
        _compute(j0, 0)

        @pl.when(k + 1 < SEQ_PER_TILE // 2)
        def _fire_next():
            pltpu.async_copy(tab_hbm.at[ids_v.at[j0 + 2]], rows_v.at[0], sem0)

        pltpu.make_async_copy(
            tab_hbm.at[ids_v.at[j1]], rows_v.at[1], sem1).wait()
        _compute(j1, 1)
        return carry

    lax.fori_loop(0, SEQ_PER_TILE // 2, pair, 0)
    pltpu.sync_copy(acc_v, out_hbm.at[wid])
    pltpu.sync_copy(corr_v, corr_hbm.at[wid])


def _make_sc_pool():
    mesh = plsc.VectorSubcoreMesh(core_axis_name="c", subcore_axis_name="s")
    return pl.kernel(
        _sc_body,
        out_type=(
            jax.ShapeDtypeStruct((32, SEQ_PER_TILE, H), jnp.float32),
            jax.ShapeDtypeStruct((32, SEQ_PER_TILE, 16), jnp.float32),
        ),
        mesh=mesh,
        compiler_params=pltpu.CompilerParams(needs_layout_passes=False),
        scratch_types=[
            pltpu.VMEM((SEQ_PER_TILE, QT), jnp.int32),
            pltpu.VMEM((QT, H), jnp.float32),
            pltpu.VMEM((2, QT, H), jnp.float32),
            pltpu.VMEM((SEQ_PER_TILE, H), jnp.float32),
            pltpu.VMEM((SEQ_PER_TILE, 16), jnp.float32),
            pltpu.VMEM((2, QT, 16), jnp.float32),
            pltpu.VMEM((QT,), jnp.float32),
            pltpu.SMEM((QT,), jnp.float32),
            pltpu.SemaphoreType.DMA,
            pltpu.SemaphoreType.DMA,
        ],
    )


def _tc_body(hp_ref, pp_ref, hc_ref, pc_ref, g_ref, bb_ref, w_ref, wb_ref,
             out_ref):
    inv = jnp.float32(1.0 / TOK)
    g = g_ref[...]
    bb = bb_ref[...]
    hp = hp_ref[...]
    pp = pp_ref[...]
    hcorr = jnp.sum(hc_ref[...], axis=(0, 2))
    pcorr = jnp.sum(pc_ref[...], axis=(0, 3))
    pooled_h = ((hp[0] + hp[1] + hp[2] + hp[3]) - hcorr[:, None]) * inv
    pooled_h = pooled_h * g + bb
    pooled_p = ((pp[0] + pp[1] + pp[2] + pp[3]) - pcorr[:, :, None]) * inv
    pooled_p = pooled_p * g + bb
    hist = lax.dot_general(pooled_h, w_ref[...], (((1,), (1,)), ((), ())),
                           precision=lax.Precision.HIGHEST,
                           preferred_element_type=jnp.float32)
    hist = hist + wb_ref[...]
    diff = pooled_p - hist[:, None, :]
    d2 = jnp.sum(diff * diff, axis=-1)
    feats = -jnp.sqrt(d2)
    m = jnp.max(feats, axis=-1, keepdims=True)
    ex = jnp.exp(feats - m)
    out_ref[...] = ex / jnp.sum(ex, axis=-1, keepdims=True)


def kernel(persona, history, word_emb, pos_emb, tok_type_emb, ln_g, ln_b, W, b):
    ids = jnp.concatenate(
        [history[:, 1:].reshape(B, TOK),
         persona[:, :, 1:].reshape(B * P, TOK)], axis=0).astype(jnp.int32)
    ids = ids.reshape(NSEQ, NQ, QT).transpose(1, 0, 2).reshape(32, SEQ_PER_TILE, QT)
    c = (pos_emb[2:2 + TOK] + tok_type_emb[0]).reshape(NQ, QT, H)

    partial, corr = _make_sc_pool()(ids, c, word_emb)
    partial = partial.reshape(NQ, NSEQ, H)
    corr = corr.reshape(NQ, NSEQ, 16)
    hp = partial[:, :B]
    pp = partial[:, B:].reshape(NQ, B, P, H)
    hc = corr[:, :B]
    pc = corr[:, B:].reshape(NQ, B, P, 16)

    return pl.pallas_call(
        _tc_body,
        out_shape=jax.ShapeDtypeStruct((B, P), jnp.float32),
    )(hp, pp, hc, pc, ln_g, ln_b, W, b)


# speedup vs baseline: 3.8866x; 2.3628x over previous
import jax
import jax.numpy as jnp
from jax import lax
from jax.experimental import pallas as pl
from jax.experimental.pallas import tpu as pltpu
from jax.experimental.pallas import tpu_sc as plsc

V = 50265
H = 768
B, P, T = 16, 8, 129
NSEQ = B + B * P
TOK = T - 1
NQ = 4
QT = TOK // NQ
SEQ_PER_TILE = NSEQ // 8
BS = 16


def _sc_gather_body(ids_hbm, tab_hbm, out_hbm, ids_v, rows_v,
                    g0, g1, w0, w1):
    wid = lax.axis_index("c") * 16 + lax.axis_index("s")
    q = wid // 8
    seq_base = (wid % 8) * SEQ_PER_TILE

    pltpu.sync_copy(ids_hbm.at[wid], ids_v)
    gsem = (g0, g1)
    wsem = (w0, w1)

    pltpu.async_copy(tab_hbm.at[ids_v.at[0]], rows_v.at[0], g0)
    pltpu.async_copy(tab_hbm.at[ids_v.at[1]], rows_v.at[1], g1)

    for j in range(SEQ_PER_TILE):
        buf = j % 2
        dst = out_hbm.at[q, seq_base + j]
        pltpu.make_async_copy(
            tab_hbm.at[ids_v.at[j]], rows_v.at[buf], gsem[buf]).wait()
        pltpu.async_copy(rows_v.at[buf], dst, wsem[buf])
        if j + 2 < SEQ_PER_TILE:
            pltpu.make_async_copy(rows_v.at[buf], dst, wsem[buf]).wait()
            pltpu.async_copy(
                tab_hbm.at[ids_v.at[j + 2]], rows_v.at[buf], gsem[buf])

    for j in (SEQ_PER_TILE - 2, SEQ_PER_TILE - 1):
        buf = j % 2
        dst = out_hbm.at[q, seq_base + j]
        pltpu.make_async_copy(rows_v.at[buf], dst, wsem[buf]).wait()


def _make_sc_gather():
    mesh = plsc.VectorSubcoreMesh(core_axis_name="c", subcore_axis_name="s")
    return pl.kernel(
        _sc_gather_body,
        out_type=jax.ShapeDtypeStruct((NQ, NSEQ, QT, H), jnp.float32),
        mesh=mesh,
        scratch_types=[
            pltpu.VMEM((SEQ_PER_TILE, QT), jnp.int32),
            pltpu.VMEM((2, QT, H), jnp.float32),
            pltpu.SemaphoreType.DMA,
            pltpu.SemaphoreType.DMA,
            pltpu.SemaphoreType.DMA,
            pltpu.SemaphoreType.DMA,
        ],
    )


def _pool_body(g_ref, c_ref, gam_ref, bet_ref, out_ref):
    e = g_ref[...] + c_ref[...][:, None]
    mu = jnp.mean(e, axis=-1, keepdims=True)
    var = jnp.mean(e * e, axis=-1, keepdims=True) - mu * mu
    w = lax.rsqrt(var + jnp.float32(1e-5))
    s = jnp.sum((e - mu) * w, axis=(0, 2))
    out_ref[...] = (s * jnp.float32(1.0 / TOK)) * gam_ref[...] + bet_ref[...]


def _tail_body(pooled_ref, w_ref, wb_ref, out_ref):
    pooled = pooled_ref[...]
    ph = pooled[:B]
    pp = pooled[B:].reshape(B, P, H)
    hist = lax.dot_general(ph, w_ref[...], (((1,), (1,)), ((), ())),
                           precision=lax.Precision.HIGHEST,
                           preferred_element_type=jnp.float32)
    hist = hist + wb_ref[...]
    diff = pp - hist[:, None, :]
    d2 = jnp.sum(diff * diff, axis=-1)
    feats = -jnp.sqrt(d2)
    m = jnp.max(feats, axis=-1, keepdims=True)
    ex = jnp.exp(feats - m)
    out_ref[...] = ex / jnp.sum(ex, axis=-1, keepdims=True)


def kernel(persona, history, word_emb, pos_emb, tok_type_emb, ln_g, ln_b, W, b):
    ids = jnp.concatenate(
        [history[:, 1:].reshape(B, TOK),
         persona[:, :, 1:].reshape(B * P, TOK)], axis=0).astype(jnp.int32)
    ids = ids.reshape(NSEQ, NQ, QT).transpose(1, 0, 2).reshape(
        32, SEQ_PER_TILE, QT)
    c = (pos_emb[2:2 + TOK] + tok_type_emb[0]).reshape(NQ, QT, H)

    gathered = _make_sc_gather()(ids, word_emb)

    pooled = pl.pallas_call(
        _pool_body,
        grid=(NSEQ // BS,),
        in_specs=[
            pl.BlockSpec((NQ, BS, QT, H), lambda i: (0, i, 0, 0)),
            pl.BlockSpec((NQ, QT, H), lambda i: (0, 0, 0)),
            pl.BlockSpec((1, H), lambda i: (0, 0)),
            pl.BlockSpec((1, H), lambda i: (0, 0)),
        ],
        out_specs=pl.BlockSpec((BS, H), lambda i: (i, 0)),
        out_shape=jax.ShapeDtypeStruct((NSEQ, H), jnp.float32),
    )(gathered, c, ln_g.reshape(1, H), ln_b.reshape(1, H))

    return pl.pallas_call(
        _tail_body,
        out_shape=jax.ShapeDtypeStruct((B, P), jnp.float32),
    )(pooled, W, b)
